# baseline (speedup 1.0000x reference)
import functools

import jax
import jax.numpy as jnp
from jax import lax
from jax.experimental import pallas as pl
from jax.experimental.pallas import tpu as pltpu

H = 2048
I = 768
E = 64
K = 8
T = 2048
P = T * K
M = 128
NB = P // M + E
PP = NB * M


def _router_body(x_ref, gw_ref, w_ref, s_ref):
    x = x_ref[...]
    gw = gw_ref[...]
    logits = lax.dot_general(x, gw, (((1,), (1,)), ((), ())),
                             preferred_element_type=jnp.float32)
    m = jnp.max(logits, axis=1, keepdims=True)
    p = jnp.exp(logits - m)
    p = p / jnp.sum(p, axis=1, keepdims=True)
    iota = lax.broadcasted_iota(jnp.int32, p.shape, 1)
    vals = []
    idxs = []
    for _ in range(K):
        mk = jnp.max(p, axis=1, keepdims=True)
        amk = jnp.min(jnp.where(p == mk, iota, E), axis=1, keepdims=True)
        vals.append(mk)
        idxs.append(amk)
        p = jnp.where(iota == amk, -1.0, p)
    w8 = jnp.concatenate(vals, axis=1)
    s8 = jnp.concatenate(idxs, axis=1)
    w_ref[...] = w8 / jnp.sum(w8, axis=1, keepdims=True)
    s_ref[...] = s8


def _route(x, gate_w, interpret=False):
    RB = 256
    return pl.pallas_call(
        _router_body,
        grid=(T // RB,),
        in_specs=[
            pl.BlockSpec((RB, H), lambda b: (b, 0)),
            pl.BlockSpec((E, H), lambda b: (0, 0)),
        ],
        out_specs=[
            pl.BlockSpec((RB, K), lambda b: (b, 0)),
            pl.BlockSpec((RB, K), lambda b: (b, 0)),
        ],
        out_shape=[
            jax.ShapeDtypeStruct((T, K), jnp.float32),
            jax.ShapeDtypeStruct((T, K), jnp.int32),
        ],
        interpret=interpret,
    )(x, gate_w)


def _plan_body(sel_ref, pos_ref, be_ref, bv_ref):
    iot = lax.broadcasted_iota(jnp.int32, (T, E), 1)
    carry = jnp.zeros((1, E), jnp.float32)
    Os = []
    incls = []
    for k in range(K):
        col = sel_ref[:, k:k + 1]
        O = (col == iot).astype(jnp.float32)
        incl = O
        sh = 1
        while sh < T:
            incl = incl + jnp.concatenate(
                [jnp.zeros((sh, E), jnp.float32), incl[:T - sh]], axis=0)
            sh *= 2
        incl = incl + carry
        Os.append(O)
        incls.append(incl)
        carry = incl[T - 1:T, :]
    counts = carry
    nblk = jnp.floor((counts + (M - 1)) / M)
    trili = (lax.broadcasted_iota(jnp.int32, (E, E), 0)
             <= lax.broadcasted_iota(jnp.int32, (E, E), 1)).astype(jnp.float32)
    cumblk = lax.dot_general(nblk, trili, (((1,), (0,)), ((), ())),
                             preferred_element_type=jnp.float32)
    pad_start = (cumblk - nblk) * M
    for k in range(K):
        slot = jnp.sum((incls[k] - 1.0 + pad_start) * Os[k],
                       axis=1, keepdims=True)
        pos_ref[:, k:k + 1] = slot.astype(jnp.int32)
    total = jnp.sum(nblk)
    biot = lax.broadcasted_iota(jnp.int32, (NB, E), 0).astype(jnp.float32)
    cb = jnp.broadcast_to(cumblk, (NB, E))
    be_raw = jnp.sum((cb <= biot).astype(jnp.int32), axis=1, keepdims=True)
    be_c = jnp.minimum(be_raw, E - 1)
    bvec = lax.broadcasted_iota(jnp.int32, (NB, 1), 0).astype(jnp.float32)
    bv = (bvec < total).astype(jnp.int32)
    last_e = jnp.sum(jnp.where(bvec == total - 1.0, be_c, 0))
    be_ref[...] = jnp.where(bv == 1, be_c, last_e)
    bv_ref[...] = bv


def _plan(sel, interpret=False):
    return pl.pallas_call(
        _plan_body,
        out_shape=[
            jax.ShapeDtypeStruct((T, K), jnp.int32),
            jax.ShapeDtypeStruct((NB, 1), jnp.int32),
            jax.ShapeDtypeStruct((NB, 1), jnp.int32),
        ],
        interpret=interpret,
    )(sel)


def _gemm_body(be_ref, bv_ref, xs_ref, gup_ref, dwn_ref, out_ref):
    b = pl.program_id(0)

    @pl.when(bv_ref[b] == 1)
    def _():
        xb = xs_ref[...].astype(jnp.bfloat16)
        w1 = gup_ref[0].astype(jnp.bfloat16)
        gu = lax.dot_general(xb, w1, (((1,), (1,)), ((), ())),
                             preferred_element_type=jnp.float32)
        g = gu[:, :I]
        u = gu[:, I:]
        act = (g / (1.0 + jnp.exp(-g))) * u
        w2 = dwn_ref[0].astype(jnp.bfloat16)
        out_ref[...] = lax.dot_general(act.astype(jnp.bfloat16), w2,
                                       (((1,), (1,)), ((), ())),
                                       preferred_element_type=jnp.float32)


def _gemm(xs, gate_up_w, down_w, be, bv, interpret=False):
    grid_spec = pltpu.PrefetchScalarGridSpec(
        num_scalar_prefetch=2,
        grid=(NB,),
        in_specs=[
            pl.BlockSpec((M, H), lambda b, be, bv: (b, 0)),
            pl.BlockSpec((1, 2 * I, H), lambda b, be, bv: (be[b], 0, 0)),
            pl.BlockSpec((1, H, I), lambda b, be, bv: (be[b], 0, 0)),
        ],
        out_specs=pl.BlockSpec((M, H), lambda b, be, bv: (b, 0)),
    )
    return pl.pallas_call(
        _gemm_body,
        grid_spec=grid_spec,
        out_shape=jax.ShapeDtypeStruct((PP, H), jnp.float32),
        compiler_params=pltpu.CompilerParams(
            dimension_semantics=("arbitrary",)),
        interpret=interpret,
    )(be, bv, xs, gate_up_w, down_w)


def kernel(hidden_states, gate_w, gate_up_w, down_w):
    x = hidden_states.reshape(T, H)
    w, sel = _route(x, gate_w)
    pos, be, bv = _plan(sel)
    be = be.reshape(NB)
    bv = bv.reshape(NB)
    tok_slot = jnp.zeros((PP,), jnp.int32).at[pos.reshape(P)].set(
        jnp.arange(P, dtype=jnp.int32) // K)
    xs = jnp.take(x, tok_slot, axis=0)
    partial = _gemm(xs, gate_up_w, down_w, be, bv)
    out = jnp.sum(w[:, :, None] * jnp.take(partial, pos.reshape(P), axis=0)
                  .reshape(T, K, H), axis=1)
    return out.reshape(1, T, H)


# speedup vs baseline: 1.1840x; 1.1840x over previous
import functools

import jax
import jax.numpy as jnp
from jax import lax
from jax.experimental import pallas as pl
from jax.experimental.pallas import tpu as pltpu

H = 2048
I = 768
E = 64
K = 8
T = 2048
P = T * K
M = 256
NB = P // M + E
PP = NB * M


def _router_body(x_ref, gw_ref, w_ref, s_ref):
    x = x_ref[...]
    gw = gw_ref[...]
    logits = lax.dot_general(x, gw, (((1,), (1,)), ((), ())),
                             preferred_element_type=jnp.float32)
    m = jnp.max(logits, axis=1, keepdims=True)
    p = jnp.exp(logits - m)
    p = p / jnp.sum(p, axis=1, keepdims=True)
    iota = lax.broadcasted_iota(jnp.int32, p.shape, 1)
    vals = []
    idxs = []
    for _ in range(K):
        mk = jnp.max(p, axis=1, keepdims=True)
        amk = jnp.min(jnp.where(p == mk, iota, E), axis=1, keepdims=True)
        vals.append(mk)
        idxs.append(amk)
        p = jnp.where(iota == amk, -1.0, p)
    w8 = jnp.concatenate(vals, axis=1)
    s8 = jnp.concatenate(idxs, axis=1)
    w_ref[...] = w8 / jnp.sum(w8, axis=1, keepdims=True)
    s_ref[...] = s8


def _route(x, gate_w, interpret=False):
    RB = 256
    return pl.pallas_call(
        _router_body,
        grid=(T // RB,),
        in_specs=[
            pl.BlockSpec((RB, H), lambda b: (b, 0)),
            pl.BlockSpec((E, H), lambda b: (0, 0)),
        ],
        out_specs=[
            pl.BlockSpec((RB, K), lambda b: (b, 0)),
            pl.BlockSpec((RB, K), lambda b: (b, 0)),
        ],
        out_shape=[
            jax.ShapeDtypeStruct((T, K), jnp.float32),
            jax.ShapeDtypeStruct((T, K), jnp.int32),
        ],
        interpret=interpret,
    )(x, gate_w)


def _plan_body(sel_ref, pos_ref, be_ref, bv_ref):
    iot = lax.broadcasted_iota(jnp.int32, (T, E), 1)
    carry = jnp.zeros((1, E), jnp.float32)
    Os = []
    incls = []
    for k in range(K):
        col = sel_ref[:, k:k + 1]
        O = (col == iot).astype(jnp.float32)
        incl = O
        sh = 1
        while sh < T:
            incl = incl + jnp.concatenate(
                [jnp.zeros((sh, E), jnp.float32), incl[:T - sh]], axis=0)
            sh *= 2
        incl = incl + carry
        Os.append(O)
        incls.append(incl)
        carry = incl[T - 1:T, :]
    counts = carry
    nblk = jnp.floor((counts + (M - 1)) / M)
    trili = (lax.broadcasted_iota(jnp.int32, (E, E), 0)
             <= lax.broadcasted_iota(jnp.int32, (E, E), 1)).astype(jnp.float32)
    cumblk = lax.dot_general(nblk, trili, (((1,), (0,)), ((), ())),
                             preferred_element_type=jnp.float32)
    pad_start = (cumblk - nblk) * M
    for k in range(K):
        slot = jnp.sum((incls[k] - 1.0 + pad_start) * Os[k],
                       axis=1, keepdims=True)
        pos_ref[:, k:k + 1] = slot.astype(jnp.int32)
    total = jnp.sum(nblk)
    biot = lax.broadcasted_iota(jnp.int32, (NB, E), 0).astype(jnp.float32)
    cb = jnp.broadcast_to(cumblk, (NB, E))
    be_raw = jnp.sum((cb <= biot).astype(jnp.int32), axis=1, keepdims=True)
    be_c = jnp.minimum(be_raw, E - 1)
    bvec = lax.broadcasted_iota(jnp.int32, (NB, 1), 0).astype(jnp.float32)
    bv = (bvec < total).astype(jnp.int32)
    last_e = jnp.sum(jnp.where(bvec == total - 1.0, be_c, 0))
    be_ref[...] = jnp.where(bv == 1, be_c, last_e)
    bv_ref[...] = bv


def _plan(sel, interpret=False):
    return pl.pallas_call(
        _plan_body,
        out_shape=[
            jax.ShapeDtypeStruct((T, K), jnp.int32),
            jax.ShapeDtypeStruct((NB, 1), jnp.int32),
            jax.ShapeDtypeStruct((NB, 1), jnp.int32),
        ],
        interpret=interpret,
    )(sel)


def _gemm_body(be_ref, bv_ref, xs_ref, gup_ref, dwn_ref, out_ref):
    b = pl.program_id(0)

    @pl.when(bv_ref[b] == 1)
    def _():
        xb = xs_ref[...]
        w1 = gup_ref[0].astype(jnp.bfloat16)
        gu = lax.dot_general(xb, w1, (((1,), (1,)), ((), ())),
                             preferred_element_type=jnp.float32)
        g = gu[:, :I]
        u = gu[:, I:]
        act = (g / (1.0 + jnp.exp(-g))) * u
        w2 = dwn_ref[0].astype(jnp.bfloat16)
        out_ref[...] = lax.dot_general(act.astype(jnp.bfloat16), w2,
                                       (((1,), (1,)), ((), ())),
                                       preferred_element_type=jnp.float32)


def _gemm(xs, gate_up_w, down_w, be, bv, interpret=False):
    grid_spec = pltpu.PrefetchScalarGridSpec(
        num_scalar_prefetch=2,
        grid=(NB,),
        in_specs=[
            pl.BlockSpec((M, H), lambda b, be, bv: (b, 0)),
            pl.BlockSpec((1, 2 * I, H), lambda b, be, bv: (be[b], 0, 0)),
            pl.BlockSpec((1, H, I), lambda b, be, bv: (be[b], 0, 0)),
        ],
        out_specs=pl.BlockSpec((M, H), lambda b, be, bv: (b, 0)),
    )
    return pl.pallas_call(
        _gemm_body,
        grid_spec=grid_spec,
        out_shape=jax.ShapeDtypeStruct((PP, H), jnp.float32),
        compiler_params=pltpu.CompilerParams(
            dimension_semantics=("arbitrary",)),
        interpret=interpret,
    )(be, bv, xs, gate_up_w, down_w)


def kernel(hidden_states, gate_w, gate_up_w, down_w):
    x = hidden_states.reshape(T, H)
    w, sel = _route(x, gate_w)
    pos, be, bv = _plan(sel)
    be = be.reshape(NB)
    bv = bv.reshape(NB)
    tok_slot = jnp.zeros((PP,), jnp.int32).at[pos.reshape(P)].set(
        jnp.arange(P, dtype=jnp.int32) // K)
    x16 = x.astype(jnp.bfloat16)
    xs = jnp.take(x16, tok_slot, axis=0)
    partial = _gemm(xs, gate_up_w, down_w, be, bv)
    out = jnp.sum(w[:, :, None] * jnp.take(partial, pos.reshape(P), axis=0)
                  .reshape(T, K, H), axis=1)
    return out.reshape(1, T, H)
